# baseline (speedup 1.0000x reference)
import jax
import jax.numpy as jnp
from jax import lax
from jax.experimental import pallas as pl

_R = 384
_S = 2048
_D = 128
_BH = 32


def _row_kernel(q_ref, k_ref, out_ref):
    q = q_ref[0]
    k = k_ref[0]
    c_row = lax.dot_general(q, k, (((1,), (1,)), ((), ())),
                            preferred_element_type=jnp.float32)
    c_col = lax.transpose(c_row[:, 1:_R + 1], (1, 0))

    j2 = lax.broadcasted_iota(jnp.int32, (_R, _S), 1)
    i2 = lax.broadcasted_iota(jnp.int32, (_R, _S), 0) + 1
    less = (c_row < c_col) | ((c_row == c_col) & (j2 < i2))
    valid = j2 >= 1
    cnt = jnp.sum(jnp.where(less & valid, 1, 0).astype(jnp.int32),
                  axis=1, keepdims=True)
    pos = cnt + 1
    hit = jnp.any(j2 == pos, axis=0, keepdims=True)
    row0 = lax.broadcasted_iota(jnp.int32, (1, _S), 1) == 0
    out_ref[0] = jnp.where(hit | row0, 1, 0).astype(jnp.int32)


def kernel(q, k):
    q0 = q[:, :1, :]
    mask_i32 = pl.pallas_call(
        _row_kernel,
        grid=(_BH,),
        in_specs=[
            pl.BlockSpec((1, 1, _D), lambda b: (b, 0, 0)),
            pl.BlockSpec((1, _S, _D), lambda b: (b, 0, 0)),
        ],
        out_specs=pl.BlockSpec((1, 1, _S), lambda b: (b, 0, 0)),
        out_shape=jax.ShapeDtypeStruct((_BH, 1, _S), jnp.int32),
    )(q0, k)
    return mask_i32[:, 0, :] != 0
